# baseline (speedup 1.0000x reference)
import functools

import jax
import jax.numpy as jnp
from jax import lax
from jax.experimental import pallas as pl
from jax.experimental.pallas import tpu as pltpu
from jax.experimental.pallas import tpu_sc as plsc

N = 10000
NP = 10240
D = 128
B = 64
EPS = 1e-5
NTILES = 32
CHUNK = 128
KC = 84
EPAD = NTILES * KC * CHUNK
ROWS_PER_TILE = NP // 16


def _sc_edge_body(zeros_hbm, h_hbm, src_hbm, dst_hbm, out_hbm,
                  src_v, dst_v, rows_a, acc, sem_a):
    c = lax.axis_index("c")
    s = lax.axis_index("s")
    tid = c * 16 + s

    pltpu.sync_copy(zeros_hbm, acc.at[pl.ds(s * ROWS_PER_TILE, ROWS_PER_TILE)])
    plsc.subcore_barrier()

    pltpu.sync_copy(src_hbm.at[tid], src_v)
    pltpu.sync_copy(dst_hbm.at[tid], dst_v)

    def step(j, carry):
        pltpu.async_copy(h_hbm.at[src_v.at[j]], rows_a, sem_a).wait()
        pltpu.sync_copy(rows_a, acc.at[dst_v.at[j]], add=True)
        return carry

    lax.fori_loop(0, KC, step, 0)
    plsc.subcore_barrier()

    pltpu.sync_copy(acc.at[pl.ds(s * ROWS_PER_TILE, ROWS_PER_TILE)],
                    out_hbm.at[c, pl.ds(s * ROWS_PER_TILE, ROWS_PER_TILE)])


def _sc_deg_body(zeros_hbm, ones_hbm, dst_hbm, out_hbm,
                 dst_v, rows_a, acc, sem_a):
    c = lax.axis_index("c")
    s = lax.axis_index("s")
    tid = c * 16 + s

    pltpu.sync_copy(zeros_hbm, acc.at[pl.ds(s * ROWS_PER_TILE, ROWS_PER_TILE)])
    pltpu.sync_copy(ones_hbm, rows_a)
    plsc.subcore_barrier()

    pltpu.sync_copy(dst_hbm.at[tid], dst_v)

    def step(j, carry):
        pltpu.sync_copy(rows_a, acc.at[dst_v.at[j]], add=True)
        return carry

    lax.fori_loop(0, KC, step, 0)
    plsc.subcore_barrier()

    pltpu.sync_copy(acc.at[pl.ds(s * ROWS_PER_TILE, ROWS_PER_TILE)],
                    out_hbm.at[c, pl.ds(s * ROWS_PER_TILE, ROWS_PER_TILE)])


def _sc_mesh():
    return plsc.VectorSubcoreMesh(core_axis_name="c", subcore_axis_name="s")


def _make_edge_pass():
    return pl.kernel(
        _sc_edge_body,
        out_type=jax.ShapeDtypeStruct((2, NP, D), jnp.float32),
        mesh=_sc_mesh(),
        scratch_types=[
            pltpu.VMEM((KC, CHUNK), jnp.int32),
            pltpu.VMEM((KC, CHUNK), jnp.int32),
            pltpu.VMEM((CHUNK, D), jnp.float32),
            pltpu.VMEM_SHARED((NP, D), jnp.float32),
            pltpu.SemaphoreType.DMA,
        ],
    )


def _make_deg_pass():
    return pl.kernel(
        _sc_deg_body,
        out_type=jax.ShapeDtypeStruct((2, NP, D), jnp.float32),
        mesh=_sc_mesh(),
        scratch_types=[
            pltpu.VMEM((KC, CHUNK), jnp.int32),
            pltpu.VMEM((CHUNK, D), jnp.float32),
            pltpu.VMEM_SHARED((NP, D), jnp.float32),
            pltpu.SemaphoreType.DMA,
        ],
    )


_RB = 512
_GRID = NP // _RB


def _t0_body(deg_ref, x_ref, w_ref, dis_ref, g_ref):
    deg = deg_ref[0] + deg_ref[1]
    dis = lax.rsqrt(jnp.maximum(deg, 1.0))
    dis_ref[...] = dis
    g_ref[...] = jnp.dot(x_ref[...], w_ref[...],
                         preferred_element_type=jnp.float32) * dis


def _combine_body(do_relu, has_prev, has_next, *refs):
    if has_prev and has_next:
        agg_ref, dis_ref, b_ref, gm_ref, bt_ref, prev_ref, w_ref, h_ref, g_ref = refs
    elif has_next:
        agg_ref, dis_ref, b_ref, gm_ref, bt_ref, w_ref, h_ref, g_ref = refs
    else:
        agg_ref, dis_ref, b_ref, gm_ref, bt_ref, prev_ref, h_ref = refs
    dis = dis_ref[...]
    z = (agg_ref[0] + agg_ref[1]) * dis + b_ref[...]
    mu = jnp.mean(z, axis=-1, keepdims=True)
    zc = z - mu
    var = jnp.mean(zc * zc, axis=-1, keepdims=True)
    h = zc * lax.rsqrt(var + EPS) * gm_ref[...] + bt_ref[...]
    if do_relu:
        h = jnp.maximum(h, 0.0)
    if has_prev:
        h = h + prev_ref[...]
    h_ref[...] = h
    if has_next:
        g_ref[...] = jnp.dot(h, w_ref[...],
                             preferred_element_type=jnp.float32) * dis


def _pool_body(h_ref, bf_ref, lw_ref, lb_ref, out_ref, sums_s, cnts_s):
    k = pl.program_id(0)

    @pl.when(k == 0)
    def _():
        sums_s[...] = jnp.zeros_like(sums_s)
        cnts_s[...] = jnp.zeros_like(cnts_s)

    bf = bf_ref[...]
    seg = lax.broadcasted_iota(jnp.int32, (_RB, D), 1).astype(jnp.float32)
    oh = (bf == seg).astype(jnp.float32)
    dn = (((0,), (0,)), ((), ()))
    sums_s[...] += lax.dot_general(oh, h_ref[...], dn,
                                   preferred_element_type=jnp.float32)
    cnts_s[...] += lax.dot_general(oh, jnp.ones((_RB, D), jnp.float32), dn,
                                   preferred_element_type=jnp.float32)

    @pl.when(k == _GRID - 1)
    def _():
        pooled = sums_s[...] / jnp.maximum(cnts_s[...], 1.0)
        out_ref[...] = jnp.dot(pooled, lw_ref[...],
                               preferred_element_type=jnp.float32) + lb_ref[...]


def _row_spec(n_lead=0):
    if n_lead:
        return pl.BlockSpec((n_lead, _RB, D), lambda k: (0, k, 0))
    return pl.BlockSpec((_RB, D), lambda k: (k, 0))


_FULL = pl.BlockSpec((D, D), lambda k: (0, 0))
_ROW1 = pl.BlockSpec((1, D), lambda k: (0, 0))


def _t0_call(deg2, xpad, w0):
    return pl.pallas_call(
        _t0_body,
        grid=(_GRID,),
        in_specs=[_row_spec(2), _row_spec(), _FULL],
        out_specs=[_row_spec(), _row_spec()],
        out_shape=[jax.ShapeDtypeStruct((NP, D), jnp.float32),
                   jax.ShapeDtypeStruct((NP, D), jnp.float32)],
    )(deg2, xpad, w0)


def _combine_call(do_relu, has_prev, has_next, agg2, dis, b, gm, bt,
                  prev=None, wnext=None):
    ins = [agg2, dis, b, gm, bt]
    specs = [_row_spec(2), _row_spec(), _ROW1, _ROW1, _ROW1]
    if has_prev:
        ins.append(prev)
        specs.append(_row_spec())
    if has_next:
        ins.append(wnext)
        specs.append(_FULL)
        out_specs = [_row_spec(), _row_spec()]
        out_shape = [jax.ShapeDtypeStruct((NP, D), jnp.float32),
                     jax.ShapeDtypeStruct((NP, D), jnp.float32)]
    else:
        out_specs = [_row_spec()]
        out_shape = [jax.ShapeDtypeStruct((NP, D), jnp.float32)]
    return pl.pallas_call(
        functools.partial(_combine_body, do_relu, has_prev, has_next),
        grid=(_GRID,),
        in_specs=specs,
        out_specs=out_specs,
        out_shape=out_shape,
    )(*ins)


def _pool_call(h, batchf, lwpad, lbbc):
    return pl.pallas_call(
        _pool_body,
        grid=(_GRID,),
        in_specs=[_row_spec(), _row_spec(), _FULL, _ROW1],
        out_specs=pl.BlockSpec((D, D), lambda k: (0, 0)),
        out_shape=jax.ShapeDtypeStruct((D, D), jnp.float32),
        scratch_shapes=[pltpu.VMEM((D, D), jnp.float32),
                        pltpu.VMEM((D, D), jnp.float32)],
    )(h, batchf, lwpad, lbbc)


def kernel(x, edge_index, batch, Ws, bs, gammas, betas, linW, linb):
    f32 = jnp.float32
    loop = jnp.arange(N, dtype=edge_index.dtype)
    src = jnp.concatenate([edge_index[0], loop])
    dst = jnp.concatenate([edge_index[1], loop])
    npad = EPAD - src.shape[0]
    src3 = jnp.concatenate([src, jnp.zeros((npad,), src.dtype)]
                           ).reshape(NTILES, KC, CHUNK)
    dst3 = jnp.concatenate(
        [dst, (N + (jnp.arange(npad) % (NP - N))).astype(dst.dtype)]
    ).reshape(NTILES, KC, CHUNK)

    xpad = jnp.zeros((NP, D), f32).at[:N].set(x)
    ones_c = jnp.ones((CHUNK, D), f32)
    zeros_t = jnp.zeros((ROWS_PER_TILE, D), f32)
    batchf = jnp.broadcast_to(
        jnp.concatenate([batch, jnp.full((NP - N,), B, batch.dtype)]
                        ).astype(f32)[:, None], (NP, D))
    lwpad = jnp.zeros((D, D), f32).at[:, :1].set(linW)
    lbbc = jnp.broadcast_to(linb.reshape(1, 1), (1, D))

    edge_pass = _make_edge_pass()

    deg2 = _make_deg_pass()(zeros_t, ones_c, dst3)
    dis, g = _t0_call(deg2, xpad, Ws[0])

    h = None
    for i in range(7):
        agg2 = edge_pass(zeros_t, g, src3, dst3)
        b_i = bs[i].reshape(1, D)
        gm_i = gammas[i].reshape(1, D)
        bt_i = betas[i].reshape(1, D)
        if i == 0:
            h, g = _combine_call(True, False, True, agg2, dis,
                                 b_i, gm_i, bt_i, wnext=Ws[1])
        elif i < 6:
            h, g = _combine_call(True, True, True, agg2, dis,
                                 b_i, gm_i, bt_i, prev=h, wnext=Ws[i + 1])
        else:
            (h,) = _combine_call(False, True, False, agg2, dis,
                                 b_i, gm_i, bt_i, prev=h)

    out = _pool_call(h, batchf, lwpad, lbbc)
    return out[:B, :1]


# speedup vs baseline: 3.1696x; 3.1696x over previous
import functools

import jax
import jax.numpy as jnp
from jax import lax
from jax.experimental import pallas as pl
from jax.experimental.pallas import tpu as pltpu
from jax.experimental.pallas import tpu_sc as plsc

N = 10000
NP = 10240
D = 128
B = 64
EPS = 1e-5
NTILES = 32
CHUNK = 128
KC = 81
EPAD = NTILES * KC * CHUNK
ROWS_PER_TILE = NP // 16


def _sc_edge_body(zeros_hbm, h_hbm, src_hbm, dst_hbm, out_hbm,
                  src_v, dst_v, rows_a, acc, sem_a):
    c = lax.axis_index("c")
    s = lax.axis_index("s")
    tid = c * 16 + s

    pltpu.sync_copy(zeros_hbm, acc.at[pl.ds(s * ROWS_PER_TILE, ROWS_PER_TILE)])
    plsc.subcore_barrier()

    pltpu.sync_copy(src_hbm.at[tid], src_v)
    pltpu.sync_copy(dst_hbm.at[tid], dst_v)

    def step(j, carry):
        pltpu.async_copy(h_hbm.at[src_v.at[j]], rows_a, sem_a).wait()
        pltpu.sync_copy(rows_a, acc.at[dst_v.at[j]], add=True)
        return carry

    lax.fori_loop(0, KC, step, 0)
    plsc.subcore_barrier()

    pltpu.sync_copy(acc.at[pl.ds(s * ROWS_PER_TILE, ROWS_PER_TILE)],
                    out_hbm.at[c, pl.ds(s * ROWS_PER_TILE, ROWS_PER_TILE)])


def _sc_deg_body(zeros_hbm, ones_hbm, dst_hbm, out_hbm,
                 dst_v, rows_a, acc, sem_a):
    c = lax.axis_index("c")
    s = lax.axis_index("s")
    tid = c * 16 + s

    pltpu.sync_copy(zeros_hbm, acc.at[pl.ds(s * ROWS_PER_TILE, ROWS_PER_TILE)])
    pltpu.sync_copy(ones_hbm, rows_a)
    plsc.subcore_barrier()

    pltpu.sync_copy(dst_hbm.at[tid], dst_v)

    def step(j, carry):
        pltpu.sync_copy(rows_a, acc.at[dst_v.at[j]], add=True)
        return carry

    lax.fori_loop(0, KC, step, 0)
    plsc.subcore_barrier()

    pltpu.sync_copy(acc.at[pl.ds(s * ROWS_PER_TILE, ROWS_PER_TILE)],
                    out_hbm.at[c, pl.ds(s * ROWS_PER_TILE, ROWS_PER_TILE)])


def _sc_mesh():
    return plsc.VectorSubcoreMesh(core_axis_name="c", subcore_axis_name="s")


def _make_edge_pass():
    return pl.kernel(
        _sc_edge_body,
        out_type=jax.ShapeDtypeStruct((2, NP, D), jnp.float32),
        mesh=_sc_mesh(),
        scratch_types=[
            pltpu.VMEM((KC, CHUNK), jnp.int32),
            pltpu.VMEM((KC, CHUNK), jnp.int32),
            pltpu.VMEM((CHUNK, D), jnp.float32),
            pltpu.VMEM_SHARED((NP, D), jnp.float32),
            pltpu.SemaphoreType.DMA,
        ],
    )


def _make_deg_pass():
    return pl.kernel(
        _sc_deg_body,
        out_type=jax.ShapeDtypeStruct((2, NP, D), jnp.float32),
        mesh=_sc_mesh(),
        scratch_types=[
            pltpu.VMEM((KC, CHUNK), jnp.int32),
            pltpu.VMEM((CHUNK, D), jnp.float32),
            pltpu.VMEM_SHARED((NP, D), jnp.float32),
            pltpu.SemaphoreType.DMA,
        ],
    )


_RB = 512
_GRID = NP // _RB


def _t0_body(deg_ref, x_ref, w_ref, dis_ref, g_ref):
    deg = deg_ref[0] + deg_ref[1]
    dis = lax.rsqrt(jnp.maximum(deg, 1.0))
    dis_ref[...] = dis
    g_ref[...] = jnp.dot(x_ref[...], w_ref[...],
                         preferred_element_type=jnp.float32) * dis


def _combine_body(do_relu, has_prev, has_next, *refs):
    if has_prev and has_next:
        agg_ref, dis_ref, b_ref, gm_ref, bt_ref, prev_ref, w_ref, h_ref, g_ref = refs
    elif has_next:
        agg_ref, dis_ref, b_ref, gm_ref, bt_ref, w_ref, h_ref, g_ref = refs
    else:
        agg_ref, dis_ref, b_ref, gm_ref, bt_ref, prev_ref, h_ref = refs
    dis = dis_ref[...]
    z = (agg_ref[0] + agg_ref[1]) * dis + b_ref[...]
    mu = jnp.mean(z, axis=-1, keepdims=True)
    zc = z - mu
    var = jnp.mean(zc * zc, axis=-1, keepdims=True)
    h = zc * lax.rsqrt(var + EPS) * gm_ref[...] + bt_ref[...]
    if do_relu:
        h = jnp.maximum(h, 0.0)
    if has_prev:
        h = h + prev_ref[...]
    h_ref[...] = h
    if has_next:
        g_ref[...] = jnp.dot(h, w_ref[...],
                             preferred_element_type=jnp.float32) * dis


def _pool_body(h_ref, bf_ref, lw_ref, lb_ref, out_ref, sums_s, cnts_s):
    k = pl.program_id(0)

    @pl.when(k == 0)
    def _():
        sums_s[...] = jnp.zeros_like(sums_s)
        cnts_s[...] = jnp.zeros_like(cnts_s)

    bf = bf_ref[...]
    seg = lax.broadcasted_iota(jnp.int32, (_RB, D), 1).astype(jnp.float32)
    oh = (bf == seg).astype(jnp.float32)
    dn = (((0,), (0,)), ((), ()))
    sums_s[...] += lax.dot_general(oh, h_ref[...], dn,
                                   preferred_element_type=jnp.float32)
    cnts_s[...] += lax.dot_general(oh, jnp.ones((_RB, D), jnp.float32), dn,
                                   preferred_element_type=jnp.float32)

    @pl.when(k == _GRID - 1)
    def _():
        pooled = sums_s[...] / jnp.maximum(cnts_s[...], 1.0)
        out_ref[...] = jnp.dot(pooled, lw_ref[...],
                               preferred_element_type=jnp.float32) + lb_ref[...]


def _row_spec(n_lead=0):
    if n_lead:
        return pl.BlockSpec((n_lead, _RB, D), lambda k: (0, k, 0))
    return pl.BlockSpec((_RB, D), lambda k: (k, 0))


_FULL = pl.BlockSpec((D, D), lambda k: (0, 0))
_ROW1 = pl.BlockSpec((1, D), lambda k: (0, 0))


def _t0_call(deg2, xpad, w0):
    return pl.pallas_call(
        _t0_body,
        grid=(_GRID,),
        in_specs=[_row_spec(2), _row_spec(), _FULL],
        out_specs=[_row_spec(), _row_spec()],
        out_shape=[jax.ShapeDtypeStruct((NP, D), jnp.float32),
                   jax.ShapeDtypeStruct((NP, D), jnp.float32)],
    )(deg2, xpad, w0)


def _combine_call(do_relu, has_prev, has_next, agg2, dis, b, gm, bt,
                  prev=None, wnext=None):
    ins = [agg2, dis, b, gm, bt]
    specs = [_row_spec(2), _row_spec(), _ROW1, _ROW1, _ROW1]
    if has_prev:
        ins.append(prev)
        specs.append(_row_spec())
    if has_next:
        ins.append(wnext)
        specs.append(_FULL)
        out_specs = [_row_spec(), _row_spec()]
        out_shape = [jax.ShapeDtypeStruct((NP, D), jnp.float32),
                     jax.ShapeDtypeStruct((NP, D), jnp.float32)]
    else:
        out_specs = [_row_spec()]
        out_shape = [jax.ShapeDtypeStruct((NP, D), jnp.float32)]
    return pl.pallas_call(
        functools.partial(_combine_body, do_relu, has_prev, has_next),
        grid=(_GRID,),
        in_specs=specs,
        out_specs=out_specs,
        out_shape=out_shape,
    )(*ins)


def _pool_call(h, batchf, lwpad, lbbc):
    return pl.pallas_call(
        _pool_body,
        grid=(_GRID,),
        in_specs=[_row_spec(), _row_spec(), _FULL, _ROW1],
        out_specs=pl.BlockSpec((D, D), lambda k: (0, 0)),
        out_shape=jax.ShapeDtypeStruct((D, D), jnp.float32),
        scratch_shapes=[pltpu.VMEM((D, D), jnp.float32),
                        pltpu.VMEM((D, D), jnp.float32)],
    )(h, batchf, lwpad, lbbc)


def kernel(x, edge_index, batch, Ws, bs, gammas, betas, linW, linb):
    f32 = jnp.float32
    loop = jnp.arange(N, dtype=edge_index.dtype)
    src = jnp.concatenate([edge_index[0], loop])
    dst = jnp.concatenate([edge_index[1], loop])
    npad = EPAD - src.shape[0]
    src3 = jnp.concatenate(
        [src, (jnp.arange(npad) % N).astype(src.dtype)]
    ).reshape(NTILES, KC, CHUNK)
    dst3 = jnp.concatenate(
        [dst, (N + (jnp.arange(npad) % (NP - N))).astype(dst.dtype)]
    ).reshape(NTILES, KC, CHUNK)

    xpad = jnp.zeros((NP, D), f32).at[:N].set(x)
    ones_c = jnp.ones((CHUNK, D), f32)
    zeros_t = jnp.zeros((ROWS_PER_TILE, D), f32)
    batchf = jnp.broadcast_to(
        jnp.concatenate([batch, jnp.full((NP - N,), B, batch.dtype)]
                        ).astype(f32)[:, None], (NP, D))
    lwpad = jnp.zeros((D, D), f32).at[:, :1].set(linW)
    lbbc = jnp.broadcast_to(linb.reshape(1, 1), (1, D))

    edge_pass = _make_edge_pass()

    deg2 = _make_deg_pass()(zeros_t, ones_c, dst3)
    dis, g = _t0_call(deg2, xpad, Ws[0])

    h = None
    for i in range(7):
        agg2 = edge_pass(zeros_t, g, src3, dst3)
        b_i = bs[i].reshape(1, D)
        gm_i = gammas[i].reshape(1, D)
        bt_i = betas[i].reshape(1, D)
        if i == 0:
            h, g = _combine_call(True, False, True, agg2, dis,
                                 b_i, gm_i, bt_i, wnext=Ws[1])
        elif i < 6:
            h, g = _combine_call(True, True, True, agg2, dis,
                                 b_i, gm_i, bt_i, prev=h, wnext=Ws[i + 1])
        else:
            (h,) = _combine_call(False, True, False, agg2, dis,
                                 b_i, gm_i, bt_i, prev=h)

    out = _pool_call(h, batchf, lwpad, lbbc)
    return out[:B, :1]


# speedup vs baseline: 3.5274x; 1.1129x over previous
import functools

import jax
import jax.numpy as jnp
from jax import lax
from jax.experimental import pallas as pl
from jax.experimental.pallas import tpu as pltpu
from jax.experimental.pallas import tpu_sc as plsc

N = 10000
NP = 10240
D = 128
B = 64
EPS = 1e-5
NTILES = 32
CHUNK = 128
KC = 84
HKC = KC // 2
EPAD = NTILES * KC * CHUNK
ROWS_PER_TILE = NP // 16


def _sc_edge_body(zeros_hbm, h_hbm, src_hbm, dst_hbm, out_hbm,
                  src_v, dst_v, rows_a, rows_b, acc,
                  sem_a, sem_b, sem_c, sem_d):
    c = lax.axis_index("c")
    s = lax.axis_index("s")
    tid = c * 16 + s

    pltpu.sync_copy(zeros_hbm, acc.at[pl.ds(s * ROWS_PER_TILE, ROWS_PER_TILE)])
    plsc.subcore_barrier()

    for half in range(2):
        pltpu.sync_copy(src_hbm.at[tid * 2 + half], src_v)
        pltpu.sync_copy(dst_hbm.at[tid * 2 + half], dst_v)

        def step(k, carry):
            j0 = 2 * k
            j1 = j0 + 1
            ga = pltpu.async_copy(h_hbm.at[src_v.at[j0]], rows_a, sem_a)
            gb = pltpu.async_copy(h_hbm.at[src_v.at[j1]], rows_b, sem_b)
            ga.wait()
            sa = pltpu.async_copy(rows_a, acc.at[dst_v.at[j0]], sem_c,
                                  add=True)
            gb.wait()
            sb = pltpu.async_copy(rows_b, acc.at[dst_v.at[j1]], sem_d,
                                  add=True)
            sa.wait()
            sb.wait()
            return carry

        lax.fori_loop(0, HKC // 2, step, 0)
    plsc.subcore_barrier()

    pltpu.sync_copy(acc.at[pl.ds(s * ROWS_PER_TILE, ROWS_PER_TILE)],
                    out_hbm.at[c, pl.ds(s * ROWS_PER_TILE, ROWS_PER_TILE)])


def _sc_deg_body(zeros_hbm, ones_hbm, dst_hbm, out_hbm,
                 dst_v, rows_a, acc, sem_a):
    c = lax.axis_index("c")
    s = lax.axis_index("s")
    tid = c * 16 + s

    pltpu.sync_copy(zeros_hbm, acc.at[pl.ds(s * ROWS_PER_TILE, ROWS_PER_TILE)])
    pltpu.sync_copy(ones_hbm, rows_a)
    plsc.subcore_barrier()

    for half in range(2):
        pltpu.sync_copy(dst_hbm.at[tid * 2 + half], dst_v)

        def step(j, carry):
            pltpu.sync_copy(rows_a, acc.at[dst_v.at[j]], add=True)
            return carry

        lax.fori_loop(0, HKC, step, 0)
    plsc.subcore_barrier()

    pltpu.sync_copy(acc.at[pl.ds(s * ROWS_PER_TILE, ROWS_PER_TILE)],
                    out_hbm.at[c, pl.ds(s * ROWS_PER_TILE, ROWS_PER_TILE)])


def _sc_mesh():
    return plsc.VectorSubcoreMesh(core_axis_name="c", subcore_axis_name="s")


def _make_edge_pass():
    return pl.kernel(
        _sc_edge_body,
        out_type=jax.ShapeDtypeStruct((2, NP, D), jnp.float32),
        mesh=_sc_mesh(),
        scratch_types=[
            pltpu.VMEM((HKC, CHUNK), jnp.int32),
            pltpu.VMEM((HKC, CHUNK), jnp.int32),
            pltpu.VMEM((CHUNK, D), jnp.float32),
            pltpu.VMEM((CHUNK, D), jnp.float32),
            pltpu.VMEM_SHARED((NP, D), jnp.float32),
            pltpu.SemaphoreType.DMA,
            pltpu.SemaphoreType.DMA,
            pltpu.SemaphoreType.DMA,
            pltpu.SemaphoreType.DMA,
        ],
    )


def _make_deg_pass():
    return pl.kernel(
        _sc_deg_body,
        out_type=jax.ShapeDtypeStruct((2, NP, D), jnp.float32),
        mesh=_sc_mesh(),
        scratch_types=[
            pltpu.VMEM((HKC, CHUNK), jnp.int32),
            pltpu.VMEM((CHUNK, D), jnp.float32),
            pltpu.VMEM_SHARED((NP, D), jnp.float32),
            pltpu.SemaphoreType.DMA,
        ],
    )


_RB = 512
_GRID = NP // _RB


def _t0_body(deg_ref, x_ref, w_ref, dis_ref, g_ref):
    deg = deg_ref[0] + deg_ref[1]
    dis = lax.rsqrt(jnp.maximum(deg, 1.0))
    dis_ref[...] = dis
    g_ref[...] = jnp.dot(x_ref[...], w_ref[...],
                         preferred_element_type=jnp.float32) * dis


def _combine_body(do_relu, has_prev, has_next, *refs):
    if has_prev and has_next:
        agg_ref, dis_ref, b_ref, gm_ref, bt_ref, prev_ref, w_ref, h_ref, g_ref = refs
    elif has_next:
        agg_ref, dis_ref, b_ref, gm_ref, bt_ref, w_ref, h_ref, g_ref = refs
    else:
        agg_ref, dis_ref, b_ref, gm_ref, bt_ref, prev_ref, h_ref = refs
    dis = dis_ref[...]
    z = (agg_ref[0] + agg_ref[1]) * dis + b_ref[...]
    mu = jnp.mean(z, axis=-1, keepdims=True)
    zc = z - mu
    var = jnp.mean(zc * zc, axis=-1, keepdims=True)
    h = zc * lax.rsqrt(var + EPS) * gm_ref[...] + bt_ref[...]
    if do_relu:
        h = jnp.maximum(h, 0.0)
    if has_prev:
        h = h + prev_ref[...]
    h_ref[...] = h
    if has_next:
        g_ref[...] = jnp.dot(h, w_ref[...],
                             preferred_element_type=jnp.float32) * dis


def _pool_body(h_ref, bf_ref, lw_ref, lb_ref, out_ref, sums_s, cnts_s):
    k = pl.program_id(0)

    @pl.when(k == 0)
    def _():
        sums_s[...] = jnp.zeros_like(sums_s)
        cnts_s[...] = jnp.zeros_like(cnts_s)

    bf = bf_ref[...]
    seg = lax.broadcasted_iota(jnp.int32, (_RB, D), 1).astype(jnp.float32)
    oh = (bf == seg).astype(jnp.float32)
    dn = (((0,), (0,)), ((), ()))
    sums_s[...] += lax.dot_general(oh, h_ref[...], dn,
                                   preferred_element_type=jnp.float32)
    cnts_s[...] += lax.dot_general(oh, jnp.ones((_RB, D), jnp.float32), dn,
                                   preferred_element_type=jnp.float32)

    @pl.when(k == _GRID - 1)
    def _():
        pooled = sums_s[...] / jnp.maximum(cnts_s[...], 1.0)
        out_ref[...] = jnp.dot(pooled, lw_ref[...],
                               preferred_element_type=jnp.float32) + lb_ref[...]


def _row_spec(n_lead=0):
    if n_lead:
        return pl.BlockSpec((n_lead, _RB, D), lambda k: (0, k, 0))
    return pl.BlockSpec((_RB, D), lambda k: (k, 0))


_FULL = pl.BlockSpec((D, D), lambda k: (0, 0))
_ROW1 = pl.BlockSpec((1, D), lambda k: (0, 0))


def _t0_call(deg2, xpad, w0):
    return pl.pallas_call(
        _t0_body,
        grid=(_GRID,),
        in_specs=[_row_spec(2), _row_spec(), _FULL],
        out_specs=[_row_spec(), _row_spec()],
        out_shape=[jax.ShapeDtypeStruct((NP, D), jnp.float32),
                   jax.ShapeDtypeStruct((NP, D), jnp.float32)],
    )(deg2, xpad, w0)


def _combine_call(do_relu, has_prev, has_next, agg2, dis, b, gm, bt,
                  prev=None, wnext=None):
    ins = [agg2, dis, b, gm, bt]
    specs = [_row_spec(2), _row_spec(), _ROW1, _ROW1, _ROW1]
    if has_prev:
        ins.append(prev)
        specs.append(_row_spec())
    if has_next:
        ins.append(wnext)
        specs.append(_FULL)
        out_specs = [_row_spec(), _row_spec()]
        out_shape = [jax.ShapeDtypeStruct((NP, D), jnp.float32),
                     jax.ShapeDtypeStruct((NP, D), jnp.float32)]
    else:
        out_specs = [_row_spec()]
        out_shape = [jax.ShapeDtypeStruct((NP, D), jnp.float32)]
    return pl.pallas_call(
        functools.partial(_combine_body, do_relu, has_prev, has_next),
        grid=(_GRID,),
        in_specs=specs,
        out_specs=out_specs,
        out_shape=out_shape,
    )(*ins)


def _pool_call(h, batchf, lwpad, lbbc):
    return pl.pallas_call(
        _pool_body,
        grid=(_GRID,),
        in_specs=[_row_spec(), _row_spec(), _FULL, _ROW1],
        out_specs=pl.BlockSpec((D, D), lambda k: (0, 0)),
        out_shape=jax.ShapeDtypeStruct((D, D), jnp.float32),
        scratch_shapes=[pltpu.VMEM((D, D), jnp.float32),
                        pltpu.VMEM((D, D), jnp.float32)],
    )(h, batchf, lwpad, lbbc)


def kernel(x, edge_index, batch, Ws, bs, gammas, betas, linW, linb):
    f32 = jnp.float32
    loop = jnp.arange(N, dtype=edge_index.dtype)
    src = jnp.concatenate([edge_index[0], loop])
    dst = jnp.concatenate([edge_index[1], loop])
    npad = EPAD - src.shape[0]
    src3 = jnp.concatenate(
        [src, (jnp.arange(npad) % N).astype(src.dtype)]
    ).reshape(NTILES * 2, HKC, CHUNK)
    dst3 = jnp.concatenate(
        [dst, (N + (jnp.arange(npad) % (NP - N))).astype(dst.dtype)]
    ).reshape(NTILES * 2, HKC, CHUNK)

    xpad = jnp.zeros((NP, D), f32).at[:N].set(x)
    ones_c = jnp.ones((CHUNK, D), f32)
    zeros_t = jnp.zeros((ROWS_PER_TILE, D), f32)
    batchf = jnp.broadcast_to(
        jnp.concatenate([batch, jnp.full((NP - N,), B, batch.dtype)]
                        ).astype(f32)[:, None], (NP, D))
    lwpad = jnp.zeros((D, D), f32).at[:, :1].set(linW)
    lbbc = jnp.broadcast_to(linb.reshape(1, 1), (1, D))

    edge_pass = _make_edge_pass()

    deg2 = _make_deg_pass()(zeros_t, ones_c, dst3)
    dis, g = _t0_call(deg2, xpad, Ws[0])

    h = None
    for i in range(7):
        agg2 = edge_pass(zeros_t, g, src3, dst3)
        b_i = bs[i].reshape(1, D)
        gm_i = gammas[i].reshape(1, D)
        bt_i = betas[i].reshape(1, D)
        if i == 0:
            h, g = _combine_call(True, False, True, agg2, dis,
                                 b_i, gm_i, bt_i, wnext=Ws[1])
        elif i < 6:
            h, g = _combine_call(True, True, True, agg2, dis,
                                 b_i, gm_i, bt_i, prev=h, wnext=Ws[i + 1])
        else:
            (h,) = _combine_call(False, True, False, agg2, dis,
                                 b_i, gm_i, bt_i, prev=h)

    out = _pool_call(h, batchf, lwpad, lbbc)
    return out[:B, :1]


# speedup vs baseline: 3.5353x; 1.0022x over previous
import functools

import jax
import jax.numpy as jnp
from jax import lax
from jax.experimental import pallas as pl
from jax.experimental.pallas import tpu as pltpu
from jax.experimental.pallas import tpu_sc as plsc

N = 10000
NP = 10240
D = 128
B = 64
EPS = 1e-5
NTILES = 32
CHUNK = 128
KC = 84
HKC = KC // 2
EPAD = NTILES * KC * CHUNK
ROWS_PER_TILE = NP // 16


def _sc_edge_body(zeros_hbm, h_hbm, src_hbm, dst_hbm, out_hbm,
                  src_v, dst_v, rows_a, rows_b, acc,
                  sem_a, sem_b, sem_c, sem_d):
    c = lax.axis_index("c")
    s = lax.axis_index("s")
    tid = c * 16 + s

    pltpu.sync_copy(src_hbm.at[tid * 2], src_v)
    pltpu.sync_copy(dst_hbm.at[tid * 2], dst_v)
    pltpu.sync_copy(zeros_hbm, acc.at[pl.ds(s * ROWS_PER_TILE, ROWS_PER_TILE)])
    plsc.subcore_barrier()

    for half in range(2):
        if half:
            pltpu.sync_copy(src_hbm.at[tid * 2 + half], src_v)
            pltpu.sync_copy(dst_hbm.at[tid * 2 + half], dst_v)

        def step(k, carry):
            j0 = 2 * k
            j1 = j0 + 1
            ga = pltpu.async_copy(h_hbm.at[src_v.at[j0]], rows_a, sem_a)
            gb = pltpu.async_copy(h_hbm.at[src_v.at[j1]], rows_b, sem_b)
            ga.wait()
            sa = pltpu.async_copy(rows_a, acc.at[dst_v.at[j0]], sem_c,
                                  add=True)
            gb.wait()
            sb = pltpu.async_copy(rows_b, acc.at[dst_v.at[j1]], sem_d,
                                  add=True)
            sa.wait()
            sb.wait()
            return carry

        lax.fori_loop(0, HKC // 2, step, 0)
    plsc.subcore_barrier()

    pltpu.sync_copy(acc.at[pl.ds(s * ROWS_PER_TILE, ROWS_PER_TILE)],
                    out_hbm.at[c, pl.ds(s * ROWS_PER_TILE, ROWS_PER_TILE)])


def _sc_deg_body(zeros_hbm, ones_hbm, dst_hbm, out_hbm,
                 dst_v, rows_a, acc, sem_c, sem_d):
    c = lax.axis_index("c")
    s = lax.axis_index("s")
    tid = c * 16 + s

    pltpu.sync_copy(dst_hbm.at[tid * 2], dst_v)
    pltpu.sync_copy(ones_hbm, rows_a)
    pltpu.sync_copy(zeros_hbm, acc.at[pl.ds(s * ROWS_PER_TILE, ROWS_PER_TILE)])
    plsc.subcore_barrier()

    for half in range(2):
        if half:
            pltpu.sync_copy(dst_hbm.at[tid * 2 + half], dst_v)

        def step(k, carry):
            j0 = 2 * k
            sa = pltpu.async_copy(rows_a, acc.at[dst_v.at[j0]], sem_c,
                                  add=True)
            sb = pltpu.async_copy(rows_a, acc.at[dst_v.at[j0 + 1]], sem_d,
                                  add=True)
            sa.wait()
            sb.wait()
            return carry

        lax.fori_loop(0, HKC // 2, step, 0)
    plsc.subcore_barrier()

    pltpu.sync_copy(acc.at[pl.ds(s * ROWS_PER_TILE, ROWS_PER_TILE)],
                    out_hbm.at[c, pl.ds(s * ROWS_PER_TILE, ROWS_PER_TILE)])


def _sc_mesh():
    return plsc.VectorSubcoreMesh(core_axis_name="c", subcore_axis_name="s")


def _make_edge_pass():
    return pl.kernel(
        _sc_edge_body,
        out_type=jax.ShapeDtypeStruct((2, NP, D), jnp.float32),
        mesh=_sc_mesh(),
        scratch_types=[
            pltpu.VMEM((HKC, CHUNK), jnp.int32),
            pltpu.VMEM((HKC, CHUNK), jnp.int32),
            pltpu.VMEM((CHUNK, D), jnp.float32),
            pltpu.VMEM((CHUNK, D), jnp.float32),
            pltpu.VMEM_SHARED((NP, D), jnp.float32),
            pltpu.SemaphoreType.DMA,
            pltpu.SemaphoreType.DMA,
            pltpu.SemaphoreType.DMA,
            pltpu.SemaphoreType.DMA,
        ],
    )


def _make_deg_pass():
    return pl.kernel(
        _sc_deg_body,
        out_type=jax.ShapeDtypeStruct((2, NP, D), jnp.float32),
        mesh=_sc_mesh(),
        scratch_types=[
            pltpu.VMEM((HKC, CHUNK), jnp.int32),
            pltpu.VMEM((CHUNK, D), jnp.float32),
            pltpu.VMEM_SHARED((NP, D), jnp.float32),
            pltpu.SemaphoreType.DMA,
            pltpu.SemaphoreType.DMA,
        ],
    )


_RB = 512
_GRID = NP // _RB


def _t0_body(deg_ref, x_ref, w_ref, dis_ref, g_ref):
    deg = deg_ref[0] + deg_ref[1]
    dis = lax.rsqrt(jnp.maximum(deg, 1.0))
    dis_ref[...] = dis
    g_ref[...] = jnp.dot(x_ref[...], w_ref[...],
                         preferred_element_type=jnp.float32) * dis


def _combine_body(do_relu, has_prev, has_next, *refs):
    if has_prev and has_next:
        agg_ref, dis_ref, b_ref, gm_ref, bt_ref, prev_ref, w_ref, h_ref, g_ref = refs
    elif has_next:
        agg_ref, dis_ref, b_ref, gm_ref, bt_ref, w_ref, h_ref, g_ref = refs
    else:
        agg_ref, dis_ref, b_ref, gm_ref, bt_ref, prev_ref, h_ref = refs
    dis = dis_ref[...]
    z = (agg_ref[0] + agg_ref[1]) * dis + b_ref[...]
    mu = jnp.mean(z, axis=-1, keepdims=True)
    zc = z - mu
    var = jnp.mean(zc * zc, axis=-1, keepdims=True)
    h = zc * lax.rsqrt(var + EPS) * gm_ref[...] + bt_ref[...]
    if do_relu:
        h = jnp.maximum(h, 0.0)
    if has_prev:
        h = h + prev_ref[...]
    h_ref[...] = h
    if has_next:
        g_ref[...] = jnp.dot(h, w_ref[...],
                             preferred_element_type=jnp.float32) * dis


def _pool_body(h_ref, bf_ref, lw_ref, lb_ref, out_ref, sums_s, cnts_s):
    k = pl.program_id(0)

    @pl.when(k == 0)
    def _():
        sums_s[...] = jnp.zeros_like(sums_s)
        cnts_s[...] = jnp.zeros_like(cnts_s)

    bf = bf_ref[...]
    seg = lax.broadcasted_iota(jnp.int32, (_RB, D), 1).astype(jnp.float32)
    oh = (bf == seg).astype(jnp.float32)
    dn = (((0,), (0,)), ((), ()))
    sums_s[...] += lax.dot_general(oh, h_ref[...], dn,
                                   preferred_element_type=jnp.float32)
    cnts_s[...] += lax.dot_general(oh, jnp.ones((_RB, D), jnp.float32), dn,
                                   preferred_element_type=jnp.float32)

    @pl.when(k == _GRID - 1)
    def _():
        pooled = sums_s[...] / jnp.maximum(cnts_s[...], 1.0)
        out_ref[...] = jnp.dot(pooled, lw_ref[...],
                               preferred_element_type=jnp.float32) + lb_ref[...]


def _row_spec(n_lead=0):
    if n_lead:
        return pl.BlockSpec((n_lead, _RB, D), lambda k: (0, k, 0))
    return pl.BlockSpec((_RB, D), lambda k: (k, 0))


_FULL = pl.BlockSpec((D, D), lambda k: (0, 0))
_ROW1 = pl.BlockSpec((1, D), lambda k: (0, 0))


def _t0_call(deg2, xpad, w0):
    return pl.pallas_call(
        _t0_body,
        grid=(_GRID,),
        in_specs=[_row_spec(2), _row_spec(), _FULL],
        out_specs=[_row_spec(), _row_spec()],
        out_shape=[jax.ShapeDtypeStruct((NP, D), jnp.float32),
                   jax.ShapeDtypeStruct((NP, D), jnp.float32)],
    )(deg2, xpad, w0)


def _combine_call(do_relu, has_prev, has_next, agg2, dis, b, gm, bt,
                  prev=None, wnext=None):
    ins = [agg2, dis, b, gm, bt]
    specs = [_row_spec(2), _row_spec(), _ROW1, _ROW1, _ROW1]
    if has_prev:
        ins.append(prev)
        specs.append(_row_spec())
    if has_next:
        ins.append(wnext)
        specs.append(_FULL)
        out_specs = [_row_spec(), _row_spec()]
        out_shape = [jax.ShapeDtypeStruct((NP, D), jnp.float32),
                     jax.ShapeDtypeStruct((NP, D), jnp.float32)]
    else:
        out_specs = [_row_spec()]
        out_shape = [jax.ShapeDtypeStruct((NP, D), jnp.float32)]
    return pl.pallas_call(
        functools.partial(_combine_body, do_relu, has_prev, has_next),
        grid=(_GRID,),
        in_specs=specs,
        out_specs=out_specs,
        out_shape=out_shape,
    )(*ins)


def _pool_call(h, batchf, lwpad, lbbc):
    return pl.pallas_call(
        _pool_body,
        grid=(_GRID,),
        in_specs=[_row_spec(), _row_spec(), _FULL, _ROW1],
        out_specs=pl.BlockSpec((D, D), lambda k: (0, 0)),
        out_shape=jax.ShapeDtypeStruct((D, D), jnp.float32),
        scratch_shapes=[pltpu.VMEM((D, D), jnp.float32),
                        pltpu.VMEM((D, D), jnp.float32)],
    )(h, batchf, lwpad, lbbc)


def kernel(x, edge_index, batch, Ws, bs, gammas, betas, linW, linb):
    f32 = jnp.float32
    loop = jnp.arange(N, dtype=edge_index.dtype)
    src = jnp.concatenate([edge_index[0], loop])
    dst = jnp.concatenate([edge_index[1], loop])
    npad = EPAD - src.shape[0]
    src3 = jnp.concatenate(
        [src, (jnp.arange(npad) % N).astype(src.dtype)]
    ).reshape(NTILES * 2, HKC, CHUNK)
    dst3 = jnp.concatenate(
        [dst, (N + (jnp.arange(npad) % (NP - N))).astype(dst.dtype)]
    ).reshape(NTILES * 2, HKC, CHUNK)

    xpad = jnp.zeros((NP, D), f32).at[:N].set(x)
    ones_c = jnp.ones((CHUNK, D), f32)
    zeros_t = jnp.zeros((ROWS_PER_TILE, D), f32)
    batchf = jnp.broadcast_to(
        jnp.concatenate([batch, jnp.full((NP - N,), B, batch.dtype)]
                        ).astype(f32)[:, None], (NP, D))
    lwpad = jnp.zeros((D, D), f32).at[:, :1].set(linW)
    lbbc = jnp.broadcast_to(linb.reshape(1, 1), (1, D))

    edge_pass = _make_edge_pass()

    deg2 = _make_deg_pass()(zeros_t, ones_c, dst3)
    dis, g = _t0_call(deg2, xpad, Ws[0])

    h = None
    for i in range(7):
        agg2 = edge_pass(zeros_t, g, src3, dst3)
        b_i = bs[i].reshape(1, D)
        gm_i = gammas[i].reshape(1, D)
        bt_i = betas[i].reshape(1, D)
        if i == 0:
            h, g = _combine_call(True, False, True, agg2, dis,
                                 b_i, gm_i, bt_i, wnext=Ws[1])
        elif i < 6:
            h, g = _combine_call(True, True, True, agg2, dis,
                                 b_i, gm_i, bt_i, prev=h, wnext=Ws[i + 1])
        else:
            (h,) = _combine_call(False, True, False, agg2, dis,
                                 b_i, gm_i, bt_i, prev=h)

    out = _pool_call(h, batchf, lwpad, lbbc)
    return out[:B, :1]
